# Initial kernel scaffold; baseline (speedup 1.0000x reference)
import jax
import jax.numpy as jnp
from jax.experimental import pallas as pl


def kernel(x, edge_index, W1, b1, W2, b2, Wl1, bl1, Wl2, bl2):
    raise NotImplementedError("write your pallas kernel here")


# speedup vs baseline: 5.4228x; 5.4228x over previous
import dataclasses
import functools

import jax
import jax.numpy as jnp
from jax import lax
from jax.experimental import pallas as pl
from jax.experimental.pallas import tpu as pltpu
from jax.experimental.pallas import tpu_sc as plsc

N = 10000
D = 128
NC = 2
NS = 16
NW = NC * NS
LANES = 16

CH = 80
N_PAD = 10240
ZROWS = 128


@functools.lru_cache(maxsize=None)
def _make_agg(E, want_counts):
    EPW = E // NW
    nchunks = EPW // CH
    rpt = N_PAD // NS

    mesh = plsc.VectorSubcoreMesh(core_axis_name="c", subcore_axis_name="s")
    out_type = [jax.ShapeDtypeStruct((NC, N_PAD, D), jnp.float32)]
    scratch = [
        pltpu.VMEM_SHARED((N_PAD, D), jnp.float32),
        pltpu.VMEM((CH,), jnp.int32),
        pltpu.VMEM((CH,), jnp.int32),
        pltpu.VMEM((CH, D), jnp.float32),
        pltpu.VMEM((ZROWS, D), jnp.float32),
    ]
    if want_counts:
        out_type.append(jax.ShapeDtypeStruct((NW, 1, N_PAD), jnp.float32))
        scratch.append(pltpu.VMEM((1, N_PAD), jnp.float32))

    def body(feat_hbm, src_hbm, dst_hbm, sums_hbm, *rest):
        if want_counts:
            cnt_hbm, acc_sh, src_v, dst_v, rows_v, stage_v, hist_v = rest
        else:
            acc_sh, src_v, dst_v, rows_v, stage_v = rest
        c = lax.axis_index("c")
        s = lax.axis_index("s")
        wid = c * NS + s

        zero_lanes = jnp.zeros((LANES,), jnp.float32)

        @pl.loop(0, ZROWS)
        def _(r):
            @pl.loop(0, D, step=LANES)
            def _(k):
                stage_v.at[r, pl.ds(k, LANES)][...] = zero_lanes

        @pl.loop(0, rpt, step=ZROWS)
        def _(r):
            pltpu.sync_copy(stage_v, acc_sh.at[pl.ds(s * rpt + r, ZROWS)])

        if want_counts:
            @pl.loop(0, N_PAD, step=LANES)
            def _(i):
                hist_v.at[0, pl.ds(i, LANES)][...] = zero_lanes

        plsc.subcore_barrier()

        ones = jnp.ones((LANES,), jnp.float32)
        zeros_i = jnp.zeros((LANES,), jnp.int32)

        @pl.loop(0, nchunks)
        def _(j):
            base = wid * EPW + j * CH
            pltpu.sync_copy(src_hbm.at[pl.ds(base, CH)], src_v)
            pltpu.sync_copy(dst_hbm.at[pl.ds(base, CH)], dst_v)
            pltpu.sync_copy(feat_hbm.at[src_v], rows_v)
            pltpu.sync_copy(rows_v, acc_sh.at[dst_v], add=True)
            if want_counts:
                @pl.loop(0, CH, step=LANES)
                def _(i):
                    idx = dst_v[pl.ds(i, LANES)]
                    plsc.addupdate_scatter(hist_v, [zeros_i, idx], ones)

        plsc.subcore_barrier()

        @pl.loop(0, rpt, step=ZROWS)
        def _(r):
            r0 = s * rpt + r
            pltpu.sync_copy(acc_sh.at[pl.ds(r0, ZROWS)], stage_v)
            pltpu.sync_copy(stage_v, sums_hbm.at[c].at[pl.ds(r0, ZROWS)])

        if want_counts:
            pltpu.sync_copy(hist_v, cnt_hbm.at[wid])

    cp = pltpu.CompilerParams()
    if "needs_layout_passes" in pltpu.CompilerParams.__dataclass_fields__:
        cp = dataclasses.replace(cp, needs_layout_passes=False)
    return pl.kernel(body, out_type=tuple(out_type), mesh=mesh,
                     scratch_types=scratch, compiler_params=cp)


def _dot(a, w):
    return lax.dot_general(a, w, (((1,), (0,)), ((), ())),
                           preferred_element_type=jnp.float32,
                           precision=lax.Precision.HIGHEST)


BN = 2048


def _combine1(x, p, cnt, wx, wm, b):
    def body(x_ref, p_ref, c_ref, wx_ref, wm_ref, b_ref, o_ref):
        counts = jnp.maximum(jnp.sum(c_ref[...], axis=0), 1.0)
        mean = (p_ref[0] + p_ref[1]) / counts[:, None]
        acc = _dot(x_ref[...], wx_ref[...]) + _dot(mean, wm_ref[...])
        o_ref[...] = jnp.maximum(acc + b_ref[...], 0.0)

    return pl.pallas_call(
        body,
        grid=(N_PAD // BN,),
        in_specs=[
            pl.BlockSpec((BN, D), lambda i: (i, 0)),
            pl.BlockSpec((NC, BN, D), lambda i: (0, i, 0)),
            pl.BlockSpec((NW, BN), lambda i: (0, i)),
            pl.BlockSpec((D, D), lambda i: (0, 0)),
            pl.BlockSpec((D, D), lambda i: (0, 0)),
            pl.BlockSpec((1, D), lambda i: (0, 0)),
        ],
        out_specs=pl.BlockSpec((BN, D), lambda i: (i, 0)),
        out_shape=jax.ShapeDtypeStruct((N, D), jnp.float32),
    )(x, p, cnt, wx, wm, b)


def _combine2(h, p, cnt, wx, wm, b, wl1, bl1, wl2, bl2):
    def body(h_ref, p_ref, c_ref, wx_ref, wm_ref, b_ref,
             wl1_ref, bl1_ref, wl2_ref, bl2_ref, o_ref):
        counts = jnp.maximum(jnp.sum(c_ref[...], axis=0), 1.0)
        mean = (p_ref[0] + p_ref[1]) / counts[:, None]
        acc = _dot(h_ref[...], wx_ref[...]) + _dot(mean, wm_ref[...])
        h2 = jnp.maximum(acc + b_ref[...], 0.0)
        t = jnp.maximum(_dot(h2, wl1_ref[...]) + bl1_ref[...], 0.0)
        o_ref[...] = _dot(t, wl2_ref[...]) + bl2_ref[...]

    return pl.pallas_call(
        body,
        grid=(N_PAD // BN,),
        in_specs=[
            pl.BlockSpec((BN, D), lambda i: (i, 0)),
            pl.BlockSpec((NC, BN, D), lambda i: (0, i, 0)),
            pl.BlockSpec((NW, BN), lambda i: (0, i)),
            pl.BlockSpec((D, D), lambda i: (0, 0)),
            pl.BlockSpec((D, D), lambda i: (0, 0)),
            pl.BlockSpec((1, D), lambda i: (0, 0)),
            pl.BlockSpec((D, 64), lambda i: (0, 0)),
            pl.BlockSpec((1, 64), lambda i: (0, 0)),
            pl.BlockSpec((64, 1), lambda i: (0, 0)),
            pl.BlockSpec((1, 1), lambda i: (0, 0)),
        ],
        out_specs=pl.BlockSpec((BN, 1), lambda i: (i, 0)),
        out_shape=jax.ShapeDtypeStruct((N, 1), jnp.float32),
    )(h, p, cnt, wx, wm, b, wl1, bl1, wl2, bl2)


def kernel(x, edge_index, W1, b1, W2, b2, Wl1, bl1, Wl2, bl2):
    E = edge_index.shape[1]
    src = edge_index[0].astype(jnp.int32)
    dst = edge_index[1].astype(jnp.int32)

    p1, cnt = _make_agg(E, True)(x, src, dst)
    cnt = cnt.reshape(NW, N_PAD)
    h1 = _combine1(x, p1, cnt, W1[:, :D].T, W1[:, D:].T, b1[None, :])
    (p2,) = _make_agg(E, False)(h1, src, dst)
    s = _combine2(h1, p2, cnt, W2[:, :D].T, W2[:, D:].T, b2[None, :],
                  Wl1.T, bl1[None, :], Wl2.T, bl2[None, :])
    return s[:, 0]


# speedup vs baseline: 6.2361x; 1.1500x over previous
import dataclasses
import functools

import jax
import jax.numpy as jnp
from jax import lax
from jax.experimental import pallas as pl
from jax.experimental.pallas import tpu as pltpu
from jax.experimental.pallas import tpu_sc as plsc

N = 10000
D = 128
NC = 2
NS = 16
NW = NC * NS
LANES = 16

CH = 128
N_PAD = 10240


def _sc_params():
    cp = pltpu.CompilerParams()
    if "needs_layout_passes" in pltpu.CompilerParams.__dataclass_fields__:
        cp = dataclasses.replace(cp, needs_layout_passes=False)
    return cp


@functools.lru_cache(maxsize=None)
def _make_agg(nchunks):
    rpt = N_PAD // NS
    mesh = plsc.VectorSubcoreMesh(core_axis_name="c", subcore_axis_name="s")
    scratch = [
        pltpu.VMEM_SHARED((N_PAD, D), jnp.float32),
        pltpu.VMEM((2, CH), jnp.int32),
        pltpu.VMEM((2, CH), jnp.int32),
        pltpu.VMEM((CH, D), jnp.float32),
        pltpu.VMEM((CH, D), jnp.float32),
        pltpu.SemaphoreType.DMA,
        pltpu.SemaphoreType.DMA,
        pltpu.SemaphoreType.DMA,
        pltpu.SemaphoreType.DMA,
    ]

    def body(feat_hbm, idx_hbm, sums_hbm,
             acc_sh, ib0, ib1, rows0, rows1, gs0, gs1, is0, is1):
        c = lax.axis_index("c")
        s = lax.axis_index("s")
        wid = c * NS + s
        my_idx = idx_hbm.at[wid]

        def idx_start(j, ib, isem):
            pltpu.async_copy(my_idx.at[j], ib, isem)

        def idx_wait(j, ib, isem):
            pltpu.make_async_copy(my_idx.at[j], ib, isem).wait()

        def gather_start(ib, rows, gsem):
            pltpu.async_copy(feat_hbm.at[ib.at[0]], rows, gsem)

        def gather_wait(ib, rows, gsem):
            pltpu.make_async_copy(feat_hbm.at[ib.at[0]], rows, gsem).wait()

        def consume(ib, rows):
            pltpu.sync_copy(rows, acc_sh.at[ib.at[1]], add=True)

        idx_start(0, ib0, is0)

        zero_lanes = jnp.zeros((LANES,), jnp.float32)

        @pl.loop(0, CH)
        def _(r):
            @pl.loop(0, D, step=LANES)
            def _(k):
                rows0.at[r, pl.ds(k, LANES)][...] = zero_lanes

        @pl.loop(0, rpt, step=CH)
        def _(r):
            pltpu.sync_copy(rows0, acc_sh.at[pl.ds(s * rpt + r, CH)])

        plsc.subcore_barrier()

        idx_wait(0, ib0, is0)
        gather_start(ib0, rows0, gs0)
        if nchunks > 1:
            idx_start(1, ib1, is1)

        def half(j, ibA, isA, rowsA, gsA, ibB, isB, rowsB, gsB):
            @pl.when(j + 1 < nchunks)
            def _():
                idx_wait(j + 1, ibB, isB)

            gather_wait(ibA, rowsA, gsA)

            @pl.when(j + 1 < nchunks)
            def _():
                gather_start(ibB, rowsB, gsB)

            consume(ibA, rowsA)

            @pl.when(j + 2 < nchunks)
            def _():
                idx_start(j + 2, ibA, isA)

        @pl.loop(0, nchunks, step=2)
        def _(j):
            half(j, ib0, is0, rows0, gs0, ib1, is1, rows1, gs1)

            @pl.when(j + 1 < nchunks)
            def _():
                half(j + 1, ib1, is1, rows1, gs1, ib0, is0, rows0, gs0)

        plsc.subcore_barrier()

        @pl.loop(0, rpt, step=CH)
        def _(r):
            r0 = s * rpt + r
            pltpu.sync_copy(acc_sh.at[pl.ds(r0, CH)], rows0)
            pltpu.sync_copy(rows0, sums_hbm.at[c].at[pl.ds(r0, CH)])

    return pl.kernel(body,
                     out_type=jax.ShapeDtypeStruct((NC, N_PAD, D), jnp.float32),
                     mesh=mesh, scratch_types=scratch,
                     compiler_params=_sc_params())


@functools.lru_cache(maxsize=None)
def _make_counts(nchunks):
    mesh = plsc.VectorSubcoreMesh(core_axis_name="c", subcore_axis_name="s")
    scratch = [
        pltpu.VMEM((nchunks, 2, CH), jnp.int32),
        pltpu.VMEM((1, N_PAD), jnp.float32),
        pltpu.SemaphoreType.DMA,
    ]

    def body(idx_hbm, cnt_hbm, idx_v, hist_v, isem):
        c = lax.axis_index("c")
        s = lax.axis_index("s")
        wid = c * NS + s
        cp = pltpu.async_copy(idx_hbm.at[wid], idx_v, isem)

        zero_lanes = jnp.zeros((LANES,), jnp.float32)

        @pl.loop(0, N_PAD, step=LANES)
        def _(i):
            hist_v.at[0, pl.ds(i, LANES)][...] = zero_lanes

        cp.wait()
        ones = jnp.ones((LANES,), jnp.float32)
        zeros_i = jnp.zeros((LANES,), jnp.int32)

        @pl.loop(0, nchunks)
        def _(j):
            @pl.loop(0, CH, step=LANES)
            def _(i):
                idx = idx_v[j, 1, pl.ds(i, LANES)]
                plsc.addupdate_scatter(hist_v, [zeros_i, idx], ones)

        pltpu.sync_copy(hist_v, cnt_hbm.at[wid])

    return pl.kernel(body,
                     out_type=jax.ShapeDtypeStruct((NW, 1, N_PAD), jnp.float32),
                     mesh=mesh, scratch_types=scratch,
                     compiler_params=_sc_params())


def _dot(a, w):
    return lax.dot_general(a, w, (((1,), (0,)), ((), ())),
                           preferred_element_type=jnp.float32,
                           precision=lax.Precision.HIGHEST)


BN = 2048


def _combine1(x, p, cnt, wx, wm, b):
    def body(x_ref, p_ref, c_ref, wx_ref, wm_ref, b_ref, o_ref):
        counts = jnp.maximum(jnp.sum(c_ref[...], axis=0), 1.0)
        mean = (p_ref[0] + p_ref[1]) / counts[:, None]
        acc = _dot(x_ref[...], wx_ref[...]) + _dot(mean, wm_ref[...])
        o_ref[...] = jnp.maximum(acc + b_ref[...], 0.0)

    return pl.pallas_call(
        body,
        grid=(N_PAD // BN,),
        in_specs=[
            pl.BlockSpec((BN, D), lambda i: (i, 0)),
            pl.BlockSpec((NC, BN, D), lambda i: (0, i, 0)),
            pl.BlockSpec((NW, BN), lambda i: (0, i)),
            pl.BlockSpec((D, D), lambda i: (0, 0)),
            pl.BlockSpec((D, D), lambda i: (0, 0)),
            pl.BlockSpec((1, D), lambda i: (0, 0)),
        ],
        out_specs=pl.BlockSpec((BN, D), lambda i: (i, 0)),
        out_shape=jax.ShapeDtypeStruct((N, D), jnp.float32),
    )(x, p, cnt, wx, wm, b)


def _combine2(h, p, cnt, wx, wm, b, wl1, bl1, wl2, bl2):
    def body(h_ref, p_ref, c_ref, wx_ref, wm_ref, b_ref,
             wl1_ref, bl1_ref, wl2_ref, bl2_ref, o_ref):
        counts = jnp.maximum(jnp.sum(c_ref[...], axis=0), 1.0)
        mean = (p_ref[0] + p_ref[1]) / counts[:, None]
        acc = _dot(h_ref[...], wx_ref[...]) + _dot(mean, wm_ref[...])
        h2 = jnp.maximum(acc + b_ref[...], 0.0)
        t = jnp.maximum(_dot(h2, wl1_ref[...]) + bl1_ref[...], 0.0)
        o_ref[...] = _dot(t, wl2_ref[...]) + bl2_ref[...]

    return pl.pallas_call(
        body,
        grid=(N_PAD // BN,),
        in_specs=[
            pl.BlockSpec((BN, D), lambda i: (i, 0)),
            pl.BlockSpec((NC, BN, D), lambda i: (0, i, 0)),
            pl.BlockSpec((NW, BN), lambda i: (0, i)),
            pl.BlockSpec((D, D), lambda i: (0, 0)),
            pl.BlockSpec((D, D), lambda i: (0, 0)),
            pl.BlockSpec((1, D), lambda i: (0, 0)),
            pl.BlockSpec((D, 64), lambda i: (0, 0)),
            pl.BlockSpec((1, 64), lambda i: (0, 0)),
            pl.BlockSpec((64, 1), lambda i: (0, 0)),
            pl.BlockSpec((1, 1), lambda i: (0, 0)),
        ],
        out_specs=pl.BlockSpec((BN, 1), lambda i: (i, 0)),
        out_shape=jax.ShapeDtypeStruct((N, 1), jnp.float32),
    )(h, p, cnt, wx, wm, b, wl1, bl1, wl2, bl2)


def kernel(x, edge_index, W1, b1, W2, b2, Wl1, bl1, Wl2, bl2):
    E = edge_index.shape[1]
    epw = E // NW
    nchunks = -(-epw // CH)
    pad = nchunks * CH - epw

    src = edge_index[0].astype(jnp.int32).reshape(NW, epw)
    dst = edge_index[1].astype(jnp.int32).reshape(NW, epw)
    src = jnp.pad(src, ((0, 0), (0, pad)), constant_values=0)
    dst = jnp.pad(dst, ((0, 0), (0, pad)), constant_values=N)
    idx = jnp.stack([src.reshape(NW, nchunks, CH),
                     dst.reshape(NW, nchunks, CH)], axis=2)

    agg = _make_agg(nchunks)
    cnt = _make_counts(nchunks)(idx)
    cnt = cnt.reshape(NW, N_PAD)
    p1 = agg(x, idx)
    h1 = _combine1(x, p1, cnt, W1[:, :D].T, W1[:, D:].T, b1[None, :])
    p2 = agg(h1, idx)
    s = _combine2(h1, p2, cnt, W2[:, :D].T, W2[:, D:].T, b2[None, :],
                  Wl1.T, bl1[None, :], Wl2.T, bl2[None, :])
    return s[:, 0]


# speedup vs baseline: 6.6427x; 1.0652x over previous
import dataclasses
import functools

import jax
import jax.numpy as jnp
from jax import lax
from jax.experimental import pallas as pl
from jax.experimental.pallas import tpu as pltpu
from jax.experimental.pallas import tpu_sc as plsc

N = 10000
D = 128
NC = 2
NS = 16
NW = NC * NS
LANES = 16

CH = 128
N_PAD = 10240


def _sc_params():
    cp = pltpu.CompilerParams()
    if "needs_layout_passes" in pltpu.CompilerParams.__dataclass_fields__:
        cp = dataclasses.replace(cp, needs_layout_passes=False)
    return cp


@functools.lru_cache(maxsize=None)
def _make_agg(nchunks):
    rpt = N_PAD // NS
    mesh = plsc.VectorSubcoreMesh(core_axis_name="c", subcore_axis_name="s")
    scratch = [
        pltpu.VMEM_SHARED((N_PAD, D), jnp.float32),
        pltpu.VMEM((2, CH), jnp.int32),
        pltpu.VMEM((2, CH), jnp.int32),
        pltpu.VMEM((CH, D), jnp.float32),
        pltpu.VMEM((CH, D), jnp.float32),
        pltpu.SemaphoreType.DMA,
        pltpu.SemaphoreType.DMA,
        pltpu.SemaphoreType.DMA,
        pltpu.SemaphoreType.DMA,
    ]

    def body(feat_hbm, idx_hbm, sums_hbm,
             acc_sh, ib0, ib1, rows0, rows1, gs0, gs1, is0, is1):
        c = lax.axis_index("c")
        s = lax.axis_index("s")
        wid = c * NS + s
        my_idx = idx_hbm.at[wid]

        def idx_start(j, ib, isem):
            pltpu.async_copy(my_idx.at[j], ib, isem)

        def idx_wait(j, ib, isem):
            pltpu.make_async_copy(my_idx.at[j], ib, isem).wait()

        def gather_start(ib, rows, gsem):
            pltpu.async_copy(feat_hbm.at[ib.at[0]], rows, gsem)

        def gather_wait(ib, rows, gsem):
            pltpu.make_async_copy(feat_hbm.at[ib.at[0]], rows, gsem).wait()

        def consume(ib, rows):
            pltpu.sync_copy(rows, acc_sh.at[ib.at[1]], add=True)

        idx_start(0, ib0, is0)

        zero_lanes = jnp.zeros((LANES,), jnp.float32)

        @pl.loop(0, CH)
        def _(r):
            @pl.loop(0, D, step=LANES)
            def _(k):
                rows0.at[r, pl.ds(k, LANES)][...] = zero_lanes

        @pl.loop(0, rpt, step=CH)
        def _(r):
            pltpu.sync_copy(rows0, acc_sh.at[pl.ds(s * rpt + r, CH)])

        plsc.subcore_barrier()

        idx_wait(0, ib0, is0)
        gather_start(ib0, rows0, gs0)
        if nchunks > 1:
            idx_start(1, ib1, is1)

        def half(j, ibA, isA, rowsA, gsA, ibB, isB, rowsB, gsB):
            @pl.when(j + 1 < nchunks)
            def _():
                idx_wait(j + 1, ibB, isB)
                gather_start(ibB, rowsB, gsB)

            gather_wait(ibA, rowsA, gsA)
            consume(ibA, rowsA)

            @pl.when(j + 2 < nchunks)
            def _():
                idx_start(j + 2, ibA, isA)

        @pl.loop(0, nchunks, step=2)
        def _(j):
            half(j, ib0, is0, rows0, gs0, ib1, is1, rows1, gs1)

            @pl.when(j + 1 < nchunks)
            def _():
                half(j + 1, ib1, is1, rows1, gs1, ib0, is0, rows0, gs0)

        plsc.subcore_barrier()

        @pl.loop(0, rpt, step=CH)
        def _(r):
            r0 = s * rpt + r
            pltpu.sync_copy(acc_sh.at[pl.ds(r0, CH)], rows0)
            pltpu.sync_copy(rows0, sums_hbm.at[c].at[pl.ds(r0, CH)])

    return pl.kernel(body,
                     out_type=jax.ShapeDtypeStruct((NC, N_PAD, D), jnp.float32),
                     mesh=mesh, scratch_types=scratch,
                     compiler_params=_sc_params())


@functools.lru_cache(maxsize=None)
def _make_counts(nchunks):
    mesh = plsc.VectorSubcoreMesh(core_axis_name="c", subcore_axis_name="s")
    scratch = [
        pltpu.VMEM((nchunks, 2, CH), jnp.int32),
        pltpu.VMEM((1, N_PAD), jnp.float32),
        pltpu.SemaphoreType.DMA,
    ]

    def body(idx_hbm, cnt_hbm, idx_v, hist_v, isem):
        c = lax.axis_index("c")
        s = lax.axis_index("s")
        wid = c * NS + s
        cp = pltpu.async_copy(idx_hbm.at[wid], idx_v, isem)

        zero_lanes = jnp.zeros((LANES,), jnp.float32)

        @pl.loop(0, N_PAD, step=LANES)
        def _(i):
            hist_v.at[0, pl.ds(i, LANES)][...] = zero_lanes

        cp.wait()
        ones = jnp.ones((LANES,), jnp.float32)
        zeros_i = jnp.zeros((LANES,), jnp.int32)

        @pl.loop(0, nchunks)
        def _(j):
            @pl.loop(0, CH, step=LANES)
            def _(i):
                idx = idx_v[j, 1, pl.ds(i, LANES)]
                plsc.addupdate_scatter(hist_v, [zeros_i, idx], ones)

        pltpu.sync_copy(hist_v, cnt_hbm.at[wid])

    return pl.kernel(body,
                     out_type=jax.ShapeDtypeStruct((NW, 1, N_PAD), jnp.float32),
                     mesh=mesh, scratch_types=scratch,
                     compiler_params=_sc_params())


def _dot(a, w):
    return lax.dot_general(a, w, (((1,), (0,)), ((), ())),
                           preferred_element_type=jnp.float32)


BN = 2048


def _combine1(x, p, cnt, wx, wm, b):
    def body(x_ref, p_ref, c_ref, wx_ref, wm_ref, b_ref, o_ref):
        counts = jnp.maximum(jnp.sum(c_ref[...], axis=0), 1.0)
        mean = (p_ref[0] + p_ref[1]) / counts[:, None]
        acc = _dot(x_ref[...], wx_ref[...]) + _dot(mean, wm_ref[...])
        o_ref[...] = jnp.maximum(acc + b_ref[...], 0.0)

    return pl.pallas_call(
        body,
        grid=(N_PAD // BN,),
        in_specs=[
            pl.BlockSpec((BN, D), lambda i: (i, 0)),
            pl.BlockSpec((NC, BN, D), lambda i: (0, i, 0)),
            pl.BlockSpec((NW, BN), lambda i: (0, i)),
            pl.BlockSpec((D, D), lambda i: (0, 0)),
            pl.BlockSpec((D, D), lambda i: (0, 0)),
            pl.BlockSpec((1, D), lambda i: (0, 0)),
        ],
        out_specs=pl.BlockSpec((BN, D), lambda i: (i, 0)),
        out_shape=jax.ShapeDtypeStruct((N, D), jnp.float32),
    )(x, p, cnt, wx, wm, b)


def _combine2(h, p, cnt, wx, wm, b, wl1, bl1, wl2, bl2):
    def body(h_ref, p_ref, c_ref, wx_ref, wm_ref, b_ref,
             wl1_ref, bl1_ref, wl2_ref, bl2_ref, o_ref):
        counts = jnp.maximum(jnp.sum(c_ref[...], axis=0), 1.0)
        mean = (p_ref[0] + p_ref[1]) / counts[:, None]
        acc = _dot(h_ref[...], wx_ref[...]) + _dot(mean, wm_ref[...])
        h2 = jnp.maximum(acc + b_ref[...], 0.0)
        t = jnp.maximum(_dot(h2, wl1_ref[...]) + bl1_ref[...], 0.0)
        o_ref[...] = _dot(t, wl2_ref[...]) + bl2_ref[...]

    return pl.pallas_call(
        body,
        grid=(N_PAD // BN,),
        in_specs=[
            pl.BlockSpec((BN, D), lambda i: (i, 0)),
            pl.BlockSpec((NC, BN, D), lambda i: (0, i, 0)),
            pl.BlockSpec((NW, BN), lambda i: (0, i)),
            pl.BlockSpec((D, D), lambda i: (0, 0)),
            pl.BlockSpec((D, D), lambda i: (0, 0)),
            pl.BlockSpec((1, D), lambda i: (0, 0)),
            pl.BlockSpec((D, 64), lambda i: (0, 0)),
            pl.BlockSpec((1, 64), lambda i: (0, 0)),
            pl.BlockSpec((64, 1), lambda i: (0, 0)),
            pl.BlockSpec((1, 1), lambda i: (0, 0)),
        ],
        out_specs=pl.BlockSpec((BN, 1), lambda i: (i, 0)),
        out_shape=jax.ShapeDtypeStruct((N, 1), jnp.float32),
    )(h, p, cnt, wx, wm, b, wl1, bl1, wl2, bl2)


def kernel(x, edge_index, W1, b1, W2, b2, Wl1, bl1, Wl2, bl2):
    E = edge_index.shape[1]
    epw = E // NW
    nchunks = -(-epw // CH)
    pad = nchunks * CH - epw

    src = edge_index[0].astype(jnp.int32).reshape(NW, epw)
    dst = edge_index[1].astype(jnp.int32).reshape(NW, epw)
    src = jnp.pad(src, ((0, 0), (0, pad)), constant_values=0)
    dst = jnp.pad(dst, ((0, 0), (0, pad)), constant_values=N)
    idx = jnp.stack([src.reshape(NW, nchunks, CH),
                     dst.reshape(NW, nchunks, CH)], axis=2)

    agg = _make_agg(nchunks)
    cnt = _make_counts(nchunks)(idx)
    cnt = cnt.reshape(NW, N_PAD)
    p1 = agg(x, idx)
    h1 = _combine1(x, p1, cnt, W1[:, :D].T, W1[:, D:].T, b1[None, :])
    p2 = agg(h1, idx)
    s = _combine2(h1, p2, cnt, W2[:, :D].T, W2[:, D:].T, b2[None, :],
                  Wl1.T, bl1[None, :], Wl2.T, bl2[None, :])
    return s[:, 0]


# speedup vs baseline: 7.7403x; 1.1652x over previous
import dataclasses
import functools

import jax
import jax.numpy as jnp
from jax import lax
from jax.experimental import pallas as pl
from jax.experimental.pallas import tpu as pltpu
from jax.experimental.pallas import tpu_sc as plsc

N = 10000
D = 128
NC = 2
NS = 16
NW = NC * NS
LANES = 16

CH = 64
NBUF = 4
N_PAD = 10240


def _sc_params():
    cp = pltpu.CompilerParams()
    if "needs_layout_passes" in pltpu.CompilerParams.__dataclass_fields__:
        cp = dataclasses.replace(cp, needs_layout_passes=False)
    return cp


@functools.lru_cache(maxsize=None)
def _make_agg(nchunks):
    rpt = N_PAD // NS
    mesh = plsc.VectorSubcoreMesh(core_axis_name="c", subcore_axis_name="s")
    assert nchunks >= NBUF
    scratch = (
        [pltpu.VMEM_SHARED((N_PAD, D), jnp.float32)]
        + [pltpu.VMEM((2, CH), jnp.int32) for _ in range(NBUF)]
        + [pltpu.VMEM((CH, D), jnp.float32) for _ in range(NBUF)]
        + [pltpu.SemaphoreType.DMA for _ in range(2 * NBUF)]
    )

    def body(feat_hbm, idx_hbm, sums_hbm, acc_sh, *rest):
        ibs = rest[:NBUF]
        rows = rest[NBUF:2 * NBUF]
        gsems = rest[2 * NBUF:3 * NBUF]
        isems = rest[3 * NBUF:4 * NBUF]
        c = lax.axis_index("c")
        s = lax.axis_index("s")
        wid = c * NS + s
        my_idx = idx_hbm.at[wid]

        def idx_start(j, k):
            pltpu.async_copy(my_idx.at[j], ibs[k], isems[k])

        def idx_wait(j, k):
            pltpu.make_async_copy(my_idx.at[j], ibs[k], isems[k]).wait()

        def gather_start(k):
            pltpu.async_copy(feat_hbm.at[ibs[k].at[0]], rows[k], gsems[k])

        def gather_wait(k):
            pltpu.make_async_copy(
                feat_hbm.at[ibs[k].at[0]], rows[k], gsems[k]).wait()

        def consume(k):
            pltpu.sync_copy(rows[k], acc_sh.at[ibs[k].at[1]], add=True)

        for k in range(NBUF):
            idx_start(k, k)

        zero_lanes = jnp.zeros((LANES,), jnp.float32)
        zbuf = rows[0]

        @pl.loop(0, CH)
        def _(r):
            @pl.loop(0, D, step=LANES)
            def _(k):
                zbuf.at[r, pl.ds(k, LANES)][...] = zero_lanes

        @pl.loop(0, rpt, step=CH)
        def _(r):
            pltpu.sync_copy(zbuf, acc_sh.at[pl.ds(s * rpt + r, CH)])

        plsc.subcore_barrier()

        for k in range(NBUF - 1):
            idx_wait(k, k)
            gather_start(k)

        @pl.loop(0, nchunks, step=NBUF)
        def _(m):
            for i in range(NBUF):
                j = m + i

                @pl.when(j < nchunks)
                def _(j=j, i=i):
                    la = NBUF - 1

                    @pl.when(j + la < nchunks)
                    def _():
                        k2 = (i + la) % NBUF
                        idx_wait(j + la, k2)
                        gather_start(k2)

                    gather_wait(i)
                    consume(i)

                    @pl.when(j + NBUF < nchunks)
                    def _():
                        idx_start(j + NBUF, i)

        plsc.subcore_barrier()

        @pl.loop(0, rpt, step=CH)
        def _(r):
            r0 = s * rpt + r
            pltpu.sync_copy(acc_sh.at[pl.ds(r0, CH)], zbuf)
            pltpu.sync_copy(zbuf, sums_hbm.at[c].at[pl.ds(r0, CH)])

    return pl.kernel(body,
                     out_type=jax.ShapeDtypeStruct((NC, N_PAD, D), jnp.float32),
                     mesh=mesh, scratch_types=scratch,
                     compiler_params=_sc_params())


@functools.lru_cache(maxsize=None)
def _make_counts(nchunks):
    mesh = plsc.VectorSubcoreMesh(core_axis_name="c", subcore_axis_name="s")
    scratch = [
        pltpu.VMEM((nchunks, 2, CH), jnp.int32),
        pltpu.VMEM((1, N_PAD), jnp.float32),
        pltpu.SemaphoreType.DMA,
    ]

    def body(idx_hbm, cnt_hbm, idx_v, hist_v, isem):
        c = lax.axis_index("c")
        s = lax.axis_index("s")
        wid = c * NS + s
        cp = pltpu.async_copy(idx_hbm.at[wid], idx_v, isem)

        zero_lanes = jnp.zeros((LANES,), jnp.float32)

        @pl.loop(0, N_PAD, step=LANES)
        def _(i):
            hist_v.at[0, pl.ds(i, LANES)][...] = zero_lanes

        cp.wait()
        ones = jnp.ones((LANES,), jnp.float32)
        zeros_i = jnp.zeros((LANES,), jnp.int32)

        @pl.loop(0, nchunks)
        def _(j):
            @pl.loop(0, CH, step=LANES)
            def _(i):
                idx = idx_v[j, 1, pl.ds(i, LANES)]
                plsc.addupdate_scatter(hist_v, [zeros_i, idx], ones)

        pltpu.sync_copy(hist_v, cnt_hbm.at[wid])

    return pl.kernel(body,
                     out_type=jax.ShapeDtypeStruct((NW, 1, N_PAD), jnp.float32),
                     mesh=mesh, scratch_types=scratch,
                     compiler_params=_sc_params())


def _dot(a, w):
    return lax.dot_general(a, w, (((1,), (0,)), ((), ())),
                           preferred_element_type=jnp.float32)


BN = 2048


def _combine1(x, p, cnt, wx, wm, b):
    def body(x_ref, p_ref, c_ref, wx_ref, wm_ref, b_ref, o_ref):
        counts = jnp.maximum(jnp.sum(c_ref[...], axis=0), 1.0)
        mean = (p_ref[0] + p_ref[1]) / counts[:, None]
        acc = _dot(x_ref[...], wx_ref[...]) + _dot(mean, wm_ref[...])
        o_ref[...] = jnp.maximum(acc + b_ref[...], 0.0)

    return pl.pallas_call(
        body,
        grid=(N_PAD // BN,),
        in_specs=[
            pl.BlockSpec((BN, D), lambda i: (i, 0)),
            pl.BlockSpec((NC, BN, D), lambda i: (0, i, 0)),
            pl.BlockSpec((NW, BN), lambda i: (0, i)),
            pl.BlockSpec((D, D), lambda i: (0, 0)),
            pl.BlockSpec((D, D), lambda i: (0, 0)),
            pl.BlockSpec((1, D), lambda i: (0, 0)),
        ],
        out_specs=pl.BlockSpec((BN, D), lambda i: (i, 0)),
        out_shape=jax.ShapeDtypeStruct((N, D), jnp.float32),
    )(x, p, cnt, wx, wm, b)


def _combine2(h, p, cnt, wx, wm, b, wl1, bl1, wl2, bl2):
    def body(h_ref, p_ref, c_ref, wx_ref, wm_ref, b_ref,
             wl1_ref, bl1_ref, wl2_ref, bl2_ref, o_ref):
        counts = jnp.maximum(jnp.sum(c_ref[...], axis=0), 1.0)
        mean = (p_ref[0] + p_ref[1]) / counts[:, None]
        acc = _dot(h_ref[...], wx_ref[...]) + _dot(mean, wm_ref[...])
        h2 = jnp.maximum(acc + b_ref[...], 0.0)
        t = jnp.maximum(_dot(h2, wl1_ref[...]) + bl1_ref[...], 0.0)
        o_ref[...] = _dot(t, wl2_ref[...]) + bl2_ref[...]

    return pl.pallas_call(
        body,
        grid=(N_PAD // BN,),
        in_specs=[
            pl.BlockSpec((BN, D), lambda i: (i, 0)),
            pl.BlockSpec((NC, BN, D), lambda i: (0, i, 0)),
            pl.BlockSpec((NW, BN), lambda i: (0, i)),
            pl.BlockSpec((D, D), lambda i: (0, 0)),
            pl.BlockSpec((D, D), lambda i: (0, 0)),
            pl.BlockSpec((1, D), lambda i: (0, 0)),
            pl.BlockSpec((D, 64), lambda i: (0, 0)),
            pl.BlockSpec((1, 64), lambda i: (0, 0)),
            pl.BlockSpec((64, 1), lambda i: (0, 0)),
            pl.BlockSpec((1, 1), lambda i: (0, 0)),
        ],
        out_specs=pl.BlockSpec((BN, 1), lambda i: (i, 0)),
        out_shape=jax.ShapeDtypeStruct((N, 1), jnp.float32),
    )(h, p, cnt, wx, wm, b, wl1, bl1, wl2, bl2)


def kernel(x, edge_index, W1, b1, W2, b2, Wl1, bl1, Wl2, bl2):
    E = edge_index.shape[1]
    epw = E // NW
    nchunks = -(-epw // CH)
    pad = nchunks * CH - epw

    src = edge_index[0].astype(jnp.int32).reshape(NW, epw)
    dst = edge_index[1].astype(jnp.int32).reshape(NW, epw)
    src = jnp.pad(src, ((0, 0), (0, pad)), constant_values=0)
    dst = jnp.pad(dst, ((0, 0), (0, pad)), constant_values=N)
    idx = jnp.stack([src.reshape(NW, nchunks, CH),
                     dst.reshape(NW, nchunks, CH)], axis=2)

    agg = _make_agg(nchunks)
    cnt = _make_counts(nchunks)(idx)
    cnt = cnt.reshape(NW, N_PAD)
    p1 = agg(x, idx)
    h1 = _combine1(x, p1, cnt, W1[:, :D].T, W1[:, D:].T, b1[None, :])
    p2 = agg(h1, idx)
    s = _combine2(h1, p2, cnt, W2[:, :D].T, W2[:, D:].T, b2[None, :],
                  Wl1.T, bl1[None, :], Wl2.T, bl2[None, :])
    return s[:, 0]


# speedup vs baseline: 7.7636x; 1.0030x over previous
import dataclasses
import functools

import jax
import jax.numpy as jnp
from jax import lax
from jax.experimental import pallas as pl
from jax.experimental.pallas import tpu as pltpu
from jax.experimental.pallas import tpu_sc as plsc

N = 10000
D = 128
NC = 2
NS = 16
NW = NC * NS
LANES = 16

CH = 64
NBUF = 5
N_PAD = 10240


def _sc_params():
    cp = pltpu.CompilerParams()
    if "needs_layout_passes" in pltpu.CompilerParams.__dataclass_fields__:
        cp = dataclasses.replace(cp, needs_layout_passes=False)
    return cp


@functools.lru_cache(maxsize=None)
def _make_agg(nchunks):
    rpt = N_PAD // NS
    mesh = plsc.VectorSubcoreMesh(core_axis_name="c", subcore_axis_name="s")
    assert nchunks >= NBUF
    scratch = (
        [pltpu.VMEM_SHARED((N_PAD, D), jnp.float32)]
        + [pltpu.VMEM((2, CH), jnp.int32) for _ in range(NBUF)]
        + [pltpu.VMEM((CH, D), jnp.float32) for _ in range(NBUF)]
        + [pltpu.SemaphoreType.DMA for _ in range(2 * NBUF)]
    )

    def body(feat_hbm, idx_hbm, sums_hbm, acc_sh, *rest):
        ibs = rest[:NBUF]
        rows = rest[NBUF:2 * NBUF]
        gsems = rest[2 * NBUF:3 * NBUF]
        isems = rest[3 * NBUF:4 * NBUF]
        c = lax.axis_index("c")
        s = lax.axis_index("s")
        wid = c * NS + s
        my_idx = idx_hbm.at[wid]

        def idx_start(j, k):
            pltpu.async_copy(my_idx.at[j], ibs[k], isems[k])

        def idx_wait(j, k):
            pltpu.make_async_copy(my_idx.at[j], ibs[k], isems[k]).wait()

        def gather_start(k):
            pltpu.async_copy(feat_hbm.at[ibs[k].at[0]], rows[k], gsems[k])

        def gather_wait(k):
            pltpu.make_async_copy(
                feat_hbm.at[ibs[k].at[0]], rows[k], gsems[k]).wait()

        def consume(k):
            pltpu.sync_copy(rows[k], acc_sh.at[ibs[k].at[1]], add=True)

        for k in range(NBUF):
            idx_start(k, k)

        zero_lanes = jnp.zeros((LANES,), jnp.float32)
        zbuf = rows[0]

        @pl.loop(0, CH)
        def _(r):
            @pl.loop(0, D, step=LANES)
            def _(k):
                zbuf.at[r, pl.ds(k, LANES)][...] = zero_lanes

        @pl.loop(0, rpt, step=CH)
        def _(r):
            pltpu.sync_copy(zbuf, acc_sh.at[pl.ds(s * rpt + r, CH)])

        plsc.subcore_barrier()

        for k in range(NBUF - 1):
            idx_wait(k, k)
            gather_start(k)

        @pl.loop(0, nchunks, step=NBUF)
        def _(m):
            for i in range(NBUF):
                j = m + i

                @pl.when(j < nchunks)
                def _(j=j, i=i):
                    la = NBUF - 1

                    @pl.when(j + la < nchunks)
                    def _():
                        k2 = (i + la) % NBUF
                        idx_wait(j + la, k2)
                        gather_start(k2)

                    gather_wait(i)
                    consume(i)

                    @pl.when(j + NBUF < nchunks)
                    def _():
                        idx_start(j + NBUF, i)

        plsc.subcore_barrier()

        @pl.loop(0, rpt, step=CH)
        def _(r):
            r0 = s * rpt + r
            pltpu.sync_copy(acc_sh.at[pl.ds(r0, CH)], zbuf)
            pltpu.sync_copy(zbuf, sums_hbm.at[c].at[pl.ds(r0, CH)])

    return pl.kernel(body,
                     out_type=jax.ShapeDtypeStruct((NC, N_PAD, D), jnp.float32),
                     mesh=mesh, scratch_types=scratch,
                     compiler_params=_sc_params())


@functools.lru_cache(maxsize=None)
def _make_counts(nchunks):
    mesh = plsc.VectorSubcoreMesh(core_axis_name="c", subcore_axis_name="s")
    scratch = [
        pltpu.VMEM((nchunks, 2, CH), jnp.int32),
        pltpu.VMEM((1, N_PAD), jnp.float32),
        pltpu.SemaphoreType.DMA,
    ]

    def body(idx_hbm, cnt_hbm, idx_v, hist_v, isem):
        c = lax.axis_index("c")
        s = lax.axis_index("s")
        wid = c * NS + s
        cp = pltpu.async_copy(idx_hbm.at[wid], idx_v, isem)

        zero_lanes = jnp.zeros((LANES,), jnp.float32)

        @pl.loop(0, N_PAD, step=LANES)
        def _(i):
            hist_v.at[0, pl.ds(i, LANES)][...] = zero_lanes

        cp.wait()
        ones = jnp.ones((LANES,), jnp.float32)
        zeros_i = jnp.zeros((LANES,), jnp.int32)

        @pl.loop(0, nchunks)
        def _(j):
            @pl.loop(0, CH, step=LANES)
            def _(i):
                idx = idx_v[j, 1, pl.ds(i, LANES)]
                plsc.addupdate_scatter(hist_v, [zeros_i, idx], ones)

        pltpu.sync_copy(hist_v, cnt_hbm.at[wid])

    return pl.kernel(body,
                     out_type=jax.ShapeDtypeStruct((NW, 1, N_PAD), jnp.float32),
                     mesh=mesh, scratch_types=scratch,
                     compiler_params=_sc_params())


def _dot(a, w):
    return lax.dot_general(a, w, (((1,), (0,)), ((), ())),
                           preferred_element_type=jnp.float32)


BN = 2048


def _combine1(x, p, cnt, wx, wm, b):
    def body(x_ref, p_ref, c_ref, wx_ref, wm_ref, b_ref, o_ref):
        counts = jnp.maximum(jnp.sum(c_ref[...], axis=0), 1.0)
        mean = (p_ref[0] + p_ref[1]) / counts[:, None]
        acc = _dot(x_ref[...], wx_ref[...]) + _dot(mean, wm_ref[...])
        o_ref[...] = jnp.maximum(acc + b_ref[...], 0.0)

    return pl.pallas_call(
        body,
        grid=(N_PAD // BN,),
        in_specs=[
            pl.BlockSpec((BN, D), lambda i: (i, 0)),
            pl.BlockSpec((NC, BN, D), lambda i: (0, i, 0)),
            pl.BlockSpec((NW, BN), lambda i: (0, i)),
            pl.BlockSpec((D, D), lambda i: (0, 0)),
            pl.BlockSpec((D, D), lambda i: (0, 0)),
            pl.BlockSpec((1, D), lambda i: (0, 0)),
        ],
        out_specs=pl.BlockSpec((BN, D), lambda i: (i, 0)),
        out_shape=jax.ShapeDtypeStruct((N, D), jnp.float32),
    )(x, p, cnt, wx, wm, b)


def _combine2(h, p, cnt, wx, wm, b, wl1, bl1, wl2, bl2):
    def body(h_ref, p_ref, c_ref, wx_ref, wm_ref, b_ref,
             wl1_ref, bl1_ref, wl2_ref, bl2_ref, o_ref):
        counts = jnp.maximum(jnp.sum(c_ref[...], axis=0), 1.0)
        mean = (p_ref[0] + p_ref[1]) / counts[:, None]
        acc = _dot(h_ref[...], wx_ref[...]) + _dot(mean, wm_ref[...])
        h2 = jnp.maximum(acc + b_ref[...], 0.0)
        t = jnp.maximum(_dot(h2, wl1_ref[...]) + bl1_ref[...], 0.0)
        o_ref[...] = _dot(t, wl2_ref[...]) + bl2_ref[...]

    return pl.pallas_call(
        body,
        grid=(N_PAD // BN,),
        in_specs=[
            pl.BlockSpec((BN, D), lambda i: (i, 0)),
            pl.BlockSpec((NC, BN, D), lambda i: (0, i, 0)),
            pl.BlockSpec((NW, BN), lambda i: (0, i)),
            pl.BlockSpec((D, D), lambda i: (0, 0)),
            pl.BlockSpec((D, D), lambda i: (0, 0)),
            pl.BlockSpec((1, D), lambda i: (0, 0)),
            pl.BlockSpec((D, 64), lambda i: (0, 0)),
            pl.BlockSpec((1, 64), lambda i: (0, 0)),
            pl.BlockSpec((64, 1), lambda i: (0, 0)),
            pl.BlockSpec((1, 1), lambda i: (0, 0)),
        ],
        out_specs=pl.BlockSpec((BN, 1), lambda i: (i, 0)),
        out_shape=jax.ShapeDtypeStruct((N, 1), jnp.float32),
    )(h, p, cnt, wx, wm, b, wl1, bl1, wl2, bl2)


def kernel(x, edge_index, W1, b1, W2, b2, Wl1, bl1, Wl2, bl2):
    E = edge_index.shape[1]
    epw = E // NW
    nchunks = -(-epw // CH)
    pad = nchunks * CH - epw

    src = edge_index[0].astype(jnp.int32).reshape(NW, epw)
    dst = edge_index[1].astype(jnp.int32).reshape(NW, epw)
    src = jnp.pad(src, ((0, 0), (0, pad)), constant_values=0)
    dst = jnp.pad(dst, ((0, 0), (0, pad)), constant_values=N)
    idx = jnp.stack([src.reshape(NW, nchunks, CH),
                     dst.reshape(NW, nchunks, CH)], axis=2)

    agg = _make_agg(nchunks)
    cnt = _make_counts(nchunks)(idx)
    cnt = cnt.reshape(NW, N_PAD)
    p1 = agg(x, idx)
    h1 = _combine1(x, p1, cnt, W1[:, :D].T, W1[:, D:].T, b1[None, :])
    p2 = agg(h1, idx)
    s = _combine2(h1, p2, cnt, W2[:, :D].T, W2[:, D:].T, b2[None, :],
                  Wl1.T, bl1[None, :], Wl2.T, bl2[None, :])
    return s[:, 0]
